# Initial kernel scaffold; baseline (speedup 1.0000x reference)
import jax
import jax.numpy as jnp
from jax.experimental import pallas as pl


def kernel(x, edge_index, u_id, pos_i_id, neg_i_id, W, att_src, att_dst, bias):
    raise NotImplementedError("write your pallas kernel here")


# speedup vs baseline: 8.2186x; 8.2186x over previous
import functools

import jax
import jax.numpy as jnp
from jax import lax
from jax.experimental import pallas as pl
from jax.experimental.pallas import tpu as pltpu
from jax.experimental.pallas import tpu_sc as plsc

N_NODES = 50000
N_EDGES = 800000
EMB = 64
B = 4096
REGS = 1e-5

NC, NS, L = 2, 16, 16
NW = NC * NS

EPAD = 819200
HALF = 25000
ACC_R = 25600
DEN_N = 51200

_R1 = 400


def _tc_h_body(x_ref, w_ref, asrc_ref, adst_ref, h_ref, as_ref, ad_ref):
    h = jnp.dot(x_ref[...], w_ref[...], preferred_element_type=jnp.float32)
    h_ref[...] = h
    as_ref[...] = jnp.sum(h * asrc_ref[...][None, :], axis=1, keepdims=True)
    ad_ref[...] = jnp.sum(h * adst_ref[...][None, :], axis=1, keepdims=True)


def _tc_h(x, W, att_src, att_dst):
    grid = (N_NODES // _R1,)
    return pl.pallas_call(
        _tc_h_body,
        grid=grid,
        in_specs=[
            pl.BlockSpec((_R1, EMB), lambda i: (i, 0)),
            pl.BlockSpec((EMB, EMB), lambda i: (0, 0)),
            pl.BlockSpec((EMB,), lambda i: (0,)),
            pl.BlockSpec((EMB,), lambda i: (0,)),
        ],
        out_specs=[
            pl.BlockSpec((_R1, EMB), lambda i: (i, 0)),
            pl.BlockSpec((_R1, 1), lambda i: (i, 0)),
            pl.BlockSpec((_R1, 1), lambda i: (i, 0)),
        ],
        out_shape=[
            jax.ShapeDtypeStruct((N_NODES, EMB), jnp.float32),
            jax.ShapeDtypeStruct((N_NODES, 1), jnp.float32),
            jax.ShapeDtypeStruct((N_NODES, 1), jnp.float32),
        ],
    )(x, W, att_src, att_dst)


_E_TEC2 = EPAD // NW
_BLK2 = 1280
_SUB2 = _BLK2 // 128


def _sc_w_body(as_h, ad_h, src_h, dst_h, zden_h,
               w_out, den_out,
               as_v, ad_v, src_v, dst_v, didx_v, wsub_v, wblk_v, den_sh):
    c = lax.axis_index("c")
    s = lax.axis_index("s")
    wid = s * NC + c
    stripe = s * (DEN_N // NS)
    pltpu.sync_copy(zden_h, den_sh.at[pl.ds(stripe, DEN_N // NS)])
    pltpu.sync_copy(as_h, as_v)
    pltpu.sync_copy(ad_h, ad_v)
    plsc.subcore_barrier()

    base_e = wid * _E_TEC2

    def outer(b, carry):
        off = base_e + b * _BLK2
        pltpu.sync_copy(src_h.at[pl.ds(off, _BLK2)], src_v)
        pltpu.sync_copy(dst_h.at[pl.ds(off, _BLK2)], dst_v)

        def sub(u, carry2):
            for k in range(8):
                o = u * 128 + k * 16
                s16 = src_v[pl.ds(o, L)]
                d16 = dst_v[pl.ds(o, L)]
                av = plsc.load_gather(as_v, [s16])
                dv = plsc.load_gather(ad_v, [d16])
                e = av + dv
                e = jnp.maximum(e, 0.2 * e)
                w = jnp.exp(e)
                wsub_v[pl.ds(k * 16, L)] = w
                didx_v[pl.ds(k * 16, L)] = d16
                wblk_v[pl.ds(o, L)] = w
            pltpu.sync_copy(wsub_v, den_sh.at[didx_v], add=True)
            return carry2

        lax.fori_loop(0, _SUB2, sub, 0)
        pltpu.sync_copy(wblk_v, w_out.at[pl.ds(off, _BLK2)])
        return carry

    lax.fori_loop(0, _E_TEC2 // _BLK2, outer, 0)
    plsc.subcore_barrier()
    pltpu.sync_copy(den_sh.at[pl.ds(stripe, DEN_N // NS)],
                    den_out.at[c, pl.ds(stripe, DEN_N // NS)])


def _sc_w(a_s, a_d, src, dst, zden):
    f = functools.partial(
        pl.kernel,
        out_type=(
            jax.ShapeDtypeStruct((EPAD,), jnp.float32),
            jax.ShapeDtypeStruct((NC, DEN_N), jnp.float32),
        ),
        mesh=plsc.VectorSubcoreMesh(core_axis_name="c", subcore_axis_name="s"),
        compiler_params=pltpu.CompilerParams(needs_layout_passes=False, use_tc_tiling_on_sc=False),
        scratch_types=[
            pltpu.VMEM((N_NODES,), jnp.float32),
            pltpu.VMEM((N_NODES,), jnp.float32),
            pltpu.VMEM((_BLK2,), jnp.int32),
            pltpu.VMEM((_BLK2,), jnp.int32),
            pltpu.VMEM((128,), jnp.int32),
            pltpu.VMEM((128,), jnp.float32),
            pltpu.VMEM((_BLK2,), jnp.float32),
            pltpu.VMEM_SHARED((DEN_N,), jnp.float32),
        ],
    )(_sc_w_body)
    return f(a_s, a_d, src, dst, zden)


_E_TEC3 = EPAD // NS
_G3 = 128
_NB3 = _E_TEC3 // _G3


def _sc_acc_body(src_h, dst_h, w_h, h2_h, zacc_h,
                 acc_out,
                 src_v, sidx_v, par_v, dst_v, il_v, w_v, rows_v, srow_v,
                 acc_sh, sem):
    c = lax.axis_index("c")
    s = lax.axis_index("s")
    rstripe = s * (ACC_R // NS)
    pltpu.sync_copy(zacc_h, acc_sh.at[pl.ds(rstripe, ACC_R // NS), :])
    plsc.subcore_barrier()

    base_e = s * _E_TEC3
    half = c * HALF

    def blk(b, carry):
        off = base_e + b * _G3
        pltpu.sync_copy(src_h.at[pl.ds(off, _G3)], src_v)
        pltpu.sync_copy(dst_h.at[pl.ds(off, _G3)], dst_v)
        pltpu.sync_copy(w_h.at[pl.ds(off, _G3)], w_v)
        for k in range(8):
            s16 = src_v[pl.ds(k * 16, L)]
            sidx_v[pl.ds(k * 16, L)] = s16 >> 1
            par_v[pl.ds(k * 16, L)] = s16 & 1
            d16 = dst_v[pl.ds(k * 16, L)]
            il = d16 - half
            ok = (il >= 0) & (il < HALF)
            il_v[pl.ds(k * 16, L)] = jnp.where(ok, il, HALF)
        pltpu.async_copy(h2_h.at[sidx_v], rows_v, sem).wait()

        def scale(e, carry2):
            ev = jnp.full((L,), e, jnp.int32)
            wspl = plsc.load_gather(w_v, [ev])
            podd = plsc.load_gather(par_v, [ev]) == 1
            for r in range(EMB // L):
                lo = rows_v[e, pl.ds(r * L, L)]
                hi = rows_v[e, pl.ds(EMB + r * L, L)]
                srow_v[e, pl.ds(r * L, L)] = jnp.where(podd, hi, lo) * wspl
            return carry2

        lax.fori_loop(0, _G3, scale, 0)
        pltpu.sync_copy(srow_v, acc_sh.at[il_v], add=True)
        return carry

    lax.fori_loop(0, _NB3, blk, 0)
    plsc.subcore_barrier()
    pltpu.sync_copy(acc_sh.at[pl.ds(rstripe, ACC_R // NS), :],
                    acc_out.at[c, pl.ds(rstripe, ACC_R // NS), :])


def _sc_acc(src, dst, w, h2, zacc):
    f = functools.partial(
        pl.kernel,
        out_type=jax.ShapeDtypeStruct((NC, ACC_R, EMB), jnp.float32),
        mesh=plsc.VectorSubcoreMesh(core_axis_name="c", subcore_axis_name="s"),
        compiler_params=pltpu.CompilerParams(needs_layout_passes=False, use_tc_tiling_on_sc=False),
        scratch_types=[
            pltpu.VMEM((_G3,), jnp.int32),
            pltpu.VMEM((_G3,), jnp.int32),
            pltpu.VMEM((_G3,), jnp.int32),
            pltpu.VMEM((_G3,), jnp.int32),
            pltpu.VMEM((_G3,), jnp.int32),
            pltpu.VMEM((_G3,), jnp.float32),
            pltpu.VMEM((_G3, 2 * EMB), jnp.float32),
            pltpu.VMEM((_G3, EMB), jnp.float32),
            pltpu.VMEM_SHARED((ACC_R, EMB), jnp.float32),
            pltpu.SemaphoreType.DMA,
        ],
    )(_sc_acc_body)
    return f(src, dst, w, h2, zacc)


def _tc_fin_body(x_ref, acc_ref, h_ref, d0_ref, d1_ref, as_ref, ad_ref, b_ref,
                 out_ref):
    e = as_ref[...] + ad_ref[...]
    wself = jnp.exp(jnp.maximum(e, 0.2 * e))
    den = d0_ref[...] + d1_ref[...] + wself + 1e-16
    gcn = ((acc_ref[...] + wself * h_ref[...]) / den) + b_ref[...][None, :]
    out_ref[...] = jnp.concatenate([x_ref[...], gcn], axis=1)


def _tc_fin(x, acc, h, den0, den1, a_s, a_d, bias):
    grid = (N_NODES // _R1,)
    return pl.pallas_call(
        _tc_fin_body,
        grid=grid,
        in_specs=[
            pl.BlockSpec((_R1, EMB), lambda i: (i, 0)),
            pl.BlockSpec((_R1, EMB), lambda i: (i, 0)),
            pl.BlockSpec((_R1, EMB), lambda i: (i, 0)),
            pl.BlockSpec((_R1, 1), lambda i: (i, 0)),
            pl.BlockSpec((_R1, 1), lambda i: (i, 0)),
            pl.BlockSpec((_R1, 1), lambda i: (i, 0)),
            pl.BlockSpec((_R1, 1), lambda i: (i, 0)),
            pl.BlockSpec((EMB,), lambda i: (0,)),
        ],
        out_specs=pl.BlockSpec((_R1, 2 * EMB), lambda i: (i, 0)),
        out_shape=jax.ShapeDtypeStruct((N_NODES, 2 * EMB), jnp.float32),
    )(x, acc, h, den0, den1, a_s, a_d, bias)


_NIDS = 3 * B
_IDS_TEC = _NIDS // NW


def _sc_gather_body(ids_h, t_h, e_out, idx_v, r_v, sem):
    c = lax.axis_index("c")
    s = lax.axis_index("s")
    wid = s * NC + c
    base = wid * _IDS_TEC
    for t in range(_IDS_TEC // 128):
        off = base + t * 128
        pltpu.sync_copy(ids_h.at[pl.ds(off, 128)], idx_v)
        pltpu.async_copy(t_h.at[idx_v], r_v, sem).wait()
        pltpu.sync_copy(r_v, e_out.at[pl.ds(off, 128), :])


def _sc_gather(ids, table):
    f = functools.partial(
        pl.kernel,
        out_type=jax.ShapeDtypeStruct((_NIDS, 2 * EMB), jnp.float32),
        mesh=plsc.VectorSubcoreMesh(core_axis_name="c", subcore_axis_name="s"),
        compiler_params=pltpu.CompilerParams(needs_layout_passes=False, use_tc_tiling_on_sc=False),
        scratch_types=[
            pltpu.VMEM((128,), jnp.int32),
            pltpu.VMEM((128, 2 * EMB), jnp.float32),
            pltpu.SemaphoreType.DMA,
        ],
    )(_sc_gather_body)
    return f(ids, table)


def _tc_score_body(ue, pe, ne, reward_ref, loss_ref, bpr_ref, reg_ref):
    uev, pev, nev = ue[...], pe[...], ne[...]
    ps = jnp.sum(uev * pev, axis=1)
    ns_ = jnp.sum(uev * nev, axis=1)
    ij = jnp.sum(nev * pev, axis=1)
    reward_ref[...] = ns_ + ij
    z = ps - ns_
    bpr = jnp.sum(jnp.log(1.0 + jnp.exp(-z))) / B
    reg = REGS * 0.5 * (jnp.sum(uev * uev) + jnp.sum(pev * pev)
                        + jnp.sum(nev * nev))
    bpr_ref[...] = jnp.reshape(bpr, (1, 1))
    reg_ref[...] = jnp.reshape(reg, (1, 1))
    loss_ref[...] = jnp.reshape(bpr + reg, (1, 1))


def _tc_score(ue, pe, ne):
    full = pl.BlockSpec((B, 2 * EMB), lambda: (0, 0))
    one = pl.BlockSpec((1, 1), lambda: (0, 0))
    return pl.pallas_call(
        _tc_score_body,
        grid=(),
        in_specs=[full] * 3,
        out_specs=[pl.BlockSpec((B,), lambda: (0,)), one, one, one],
        out_shape=[
            jax.ShapeDtypeStruct((B,), jnp.float32),
            jax.ShapeDtypeStruct((1, 1), jnp.float32),
            jax.ShapeDtypeStruct((1, 1), jnp.float32),
            jax.ShapeDtypeStruct((1, 1), jnp.float32),
        ],
    )(ue, pe, ne)


def kernel(x, edge_index, u_id, pos_i_id, neg_i_id, W, att_src, att_dst, bias):
    x = x.astype(jnp.float32)
    src = edge_index[0].astype(jnp.int32)
    dst = edge_index[1].astype(jnp.int32)
    npad = EPAD - N_EDGES
    src = jnp.concatenate([src, jnp.zeros((npad,), jnp.int32)])
    dst = jnp.concatenate([dst, jnp.full((npad,), N_NODES, jnp.int32)])

    h, a_s, a_d = _tc_h(x, W, att_src, att_dst)

    zden = jnp.zeros((DEN_N // NS,), jnp.float32)
    w_e, den = _sc_w(a_s.reshape(-1), a_d.reshape(-1), src, dst, zden)

    zacc = jnp.zeros((ACC_R // NS, EMB), jnp.float32)
    acc = _sc_acc(src, dst, w_e, h.reshape(N_NODES // 2, 2 * EMB), zacc)

    acc_cat = jnp.concatenate([acc[0, :HALF], acc[1, :HALF]], axis=0)
    table = _tc_fin(x, acc_cat, h,
                    den[0, :N_NODES].reshape(-1, 1),
                    den[1, :N_NODES].reshape(-1, 1),
                    a_s, a_d, bias)

    ids = jnp.concatenate([u_id, pos_i_id, neg_i_id]).astype(jnp.int32)
    emb = _sc_gather(ids, table)

    ue, pe, ne = emb[:B], emb[B:2 * B], emb[2 * B:]
    reward, loss, bpr_loss, reg_loss = _tc_score(ue, pe, ne)
    return (reward, loss.reshape(()), bpr_loss.reshape(()), reg_loss.reshape(()))
